# Initial kernel scaffold; baseline (speedup 1.0000x reference)
import functools

import jax
import jax.numpy as jnp
from jax import lax
from jax.experimental import pallas as pl
from jax.experimental.pallas import tpu as pltpu
from jax.experimental.pallas import tpu_sc as plsc

_NC = 2
_NS = 16
_K = 80


def _sc_aggregate(h, src3, dst3):
    N, D = h.shape
    NW, C, K = src3.shape
    ROWS = N // _NS
    ZCH = 5
    ZR = ROWS // ZCH

    mesh = plsc.VectorSubcoreMesh(core_axis_name="c", subcore_axis_name="s")

    @functools.partial(
        pl.kernel,
        out_type=jax.ShapeDtypeStruct((_NC, N, D), jnp.float32),
        mesh=mesh,
        scratch_types=[
            pltpu.VMEM_SHARED((N, D), jnp.float32),
            pltpu.VMEM((C, K), jnp.int32),
            pltpu.VMEM((C, K), jnp.int32),
            pltpu.VMEM((K, D), jnp.float32),
            pltpu.VMEM((ZR, D), jnp.float32),
            pltpu.SemaphoreType.DMA,
        ],
    )
    def agg_kernel(h_hbm, src_hbm, dst_hbm, out_hbm,
                   agg_sh, src_v, dst_v, rows_v, zero_v, sem):
        cid = lax.axis_index("c")
        sid = lax.axis_index("s")
        wid = cid * _NS + sid

        zvec = jnp.zeros((16,), jnp.float32)

        def zfill(r, carry):
            for q in range(D // 16):
                zero_v[r, pl.ds(q * 16, 16)] = zvec
            return carry

        lax.fori_loop(0, ZR, zfill, 0)

        base = sid * ROWS

        def zcopy(i, carry):
            pltpu.sync_copy(zero_v, agg_sh.at[pl.ds(base + i * ZR, ZR)])
            return carry

        lax.fori_loop(0, ZCH, zcopy, 0)
        plsc.subcore_barrier()

        pltpu.sync_copy(src_hbm.at[wid], src_v)
        pltpu.sync_copy(dst_hbm.at[wid], dst_v)

        def ebody(j, carry):
            pltpu.async_copy(h_hbm.at[src_v.at[j]], rows_v, sem).wait()
            pltpu.sync_copy(rows_v, agg_sh.at[dst_v.at[j]], add=True)
            return carry

        lax.fori_loop(0, C, ebody, 0)
        plsc.subcore_barrier()

        pltpu.sync_copy(agg_sh.at[pl.ds(base, ROWS)],
                        out_hbm.at[cid, pl.ds(base, ROWS)])

    return agg_kernel(h, src3, dst3)


def _row_spec(D):
    return pl.BlockSpec((1, D), lambda i: (0, 0))


def _mat_spec(D):
    return pl.BlockSpec((D, D), lambda i: (0, 0))


def _gin_mlp(z, wa, ba, g, be, rm, rv, wb, bb):
    z = jnp.dot(z, wa[...], preferred_element_type=jnp.float32) + ba[...]
    scale = g[...] * lax.rsqrt(rv[...] + 1e-5)
    z = (z - rm[...]) * scale + be[...]
    z = jnp.maximum(z, 0.0)
    z = jnp.dot(z, wb[...], preferred_element_type=jnp.float32) + bb[...]
    return jnp.maximum(z, 0.0)


def _tc_layer1(h, a0, a1, Wa, ba, g, be, rm, rv, Wb, bb, blk=2000):
    N, D = h.shape
    grid = N // blk

    def body(h_ref, a0_ref, a1_ref, wa, ba_r, g_r, be_r, rm_r, rv_r, wb, bb_r,
             o_ref):
        z = h_ref[...] + a0_ref[...] + a1_ref[...]
        o_ref[...] = _gin_mlp(z, wa, ba_r, g_r, be_r, rm_r, rv_r, wb, bb_r)

    return pl.pallas_call(
        body,
        grid=(grid,),
        in_specs=[pl.BlockSpec((blk, D), lambda i: (i, 0))] * 3
        + [_mat_spec(D)] + [_row_spec(D)] * 5 + [_mat_spec(D), _row_spec(D)],
        out_specs=pl.BlockSpec((blk, D), lambda i: (i, 0)),
        out_shape=jax.ShapeDtypeStruct((N, D), jnp.float32),
    )(h, a0, a1, Wa, ba.reshape(1, D), g.reshape(1, D), be.reshape(1, D),
      rm.reshape(1, D), rv.reshape(1, D), Wb, bb.reshape(1, D))


def _tc_layer2_head(h, a0, a1, Wa, ba, g, be, rm, rv, Wb, bb,
                    W5, b5, W6, b6, blk=2000):
    N, D = h.shape
    OUT = W6.shape[1]
    grid = N // blk

    def body(h_ref, a0_ref, a1_ref, wa, ba_r, g_r, be_r, rm_r, rv_r, wb, bb_r,
             w5, b5_r, w6, b6_r, o_ref, acc):
        i = pl.program_id(0)
        z = h_ref[...] + a0_ref[...] + a1_ref[...]
        z = _gin_mlp(z, wa, ba_r, g_r, be_r, rm_r, rv_r, wb, bb_r)
        psum = jnp.sum(z, axis=0, keepdims=True)

        @pl.when(i == 0)
        def _():
            acc[...] = psum

        @pl.when(i > 0)
        def _():
            acc[...] = acc[...] + psum

        @pl.when(i == grid - 1)
        def _():
            y = jnp.dot(acc[...], w5[...], preferred_element_type=jnp.float32)
            y = jnp.maximum(y + b5_r[...], 0.0)
            o_ref[...] = (jnp.dot(y, w6[...], preferred_element_type=jnp.float32)
                          + b6_r[...])

    return pl.pallas_call(
        body,
        grid=(grid,),
        in_specs=[pl.BlockSpec((blk, D), lambda i: (i, 0))] * 3
        + [_mat_spec(D)] + [_row_spec(D)] * 5 + [_mat_spec(D), _row_spec(D)]
        + [pl.BlockSpec((D, OUT), lambda i: (0, 0)),
           pl.BlockSpec((1, OUT), lambda i: (0, 0)),
           pl.BlockSpec((D, OUT), lambda i: (0, 0)),
           pl.BlockSpec((1, OUT), lambda i: (0, 0))],
        out_specs=pl.BlockSpec((1, OUT), lambda i: (0, 0)),
        out_shape=jax.ShapeDtypeStruct((1, OUT), jnp.float32),
        scratch_shapes=[pltpu.VMEM((1, D), jnp.float32)],
    )(h, a0, a1, Wa, ba.reshape(1, D), g.reshape(1, D), be.reshape(1, D),
      rm.reshape(1, D), rv.reshape(1, D), Wb, bb.reshape(1, D),
      W5, b5.reshape(1, OUT), W6, b6.reshape(1, OUT))


def kernel(x, edge_index, W1, b1, g1, be1, W2, b2, W3, b3, g2, be2,
           W4, b4, W5, b5, W6, b6, rm1, rv1, rm2, rv2):
    N, D = x.shape
    E = edge_index.shape[1]
    NW = _NC * _NS
    C = E // (NW * _K)
    src3 = edge_index[0].reshape(NW, C, _K)
    dst3 = edge_index[1].reshape(NW, C, _K)

    p1 = _sc_aggregate(x, src3, dst3)
    h1 = _tc_layer1(x, p1[0], p1[1], W1, b1, g1, be1, rm1, rv1, W2, b2)
    p2 = _sc_aggregate(h1, src3, dst3)
    return _tc_layer2_head(h1, p2[0], p2[1], W3, b3, g2, be2, rm2, rv2,
                           W4, b4, W5, b5, W6, b6)


# speedup vs baseline: 6.6870x; 6.6870x over previous
import functools

import jax
import jax.numpy as jnp
from jax import lax
from jax.experimental import pallas as pl
from jax.experimental.pallas import tpu as pltpu
from jax.experimental.pallas import tpu_sc as plsc

_NC = 2
_NS = 16
_K = 80


def _sc_aggregate(h, src3, dst3, zeros_np):
    N, D = h.shape
    NW, C, K = src3.shape
    ROWS = -(-N // (_NS * 8)) * 8
    NP = ROWS * _NS

    mesh = plsc.VectorSubcoreMesh(core_axis_name="c", subcore_axis_name="s")

    @functools.partial(
        pl.kernel,
        out_type=jax.ShapeDtypeStruct((_NC, NP, D), jnp.float32),
        mesh=mesh,
        scratch_types=[
            pltpu.VMEM_SHARED((NP, D), jnp.float32),
            pltpu.VMEM((C, K), jnp.int32),
            pltpu.VMEM((C, K), jnp.int32),
            pltpu.VMEM((K, D), jnp.float32),
            pltpu.SemaphoreType.DMA,
        ],
    )
    def agg_kernel(h_hbm, src_hbm, dst_hbm, zeros_hbm, out_hbm,
                   agg_sh, src_v, dst_v, rows_v, sem):
        cid = lax.axis_index("c")
        sid = lax.axis_index("s")
        wid = cid * _NS + sid

        base = sid * ROWS
        pltpu.sync_copy(zeros_hbm.at[pl.ds(base, ROWS)],
                        agg_sh.at[pl.ds(base, ROWS)])
        plsc.subcore_barrier()

        pltpu.sync_copy(src_hbm.at[wid], src_v)
        pltpu.sync_copy(dst_hbm.at[wid], dst_v)

        def ebody(j, carry):
            pltpu.async_copy(h_hbm.at[src_v.at[j]], rows_v, sem).wait()
            pltpu.sync_copy(rows_v, agg_sh.at[dst_v.at[j]], add=True)
            return carry

        lax.fori_loop(0, C, ebody, 0)
        plsc.subcore_barrier()

        pltpu.sync_copy(agg_sh.at[pl.ds(base, ROWS)],
                        out_hbm.at[cid, pl.ds(base, ROWS)])

    return agg_kernel(h, src3, dst3, zeros_np)


def _row_spec(D):
    return pl.BlockSpec((1, D), lambda i: (0, 0))


def _mat_spec(D):
    return pl.BlockSpec((D, D), lambda i: (0, 0))


def _dot(a, b):
    return jnp.dot(a, b, preferred_element_type=jnp.float32)


def _gin_mlp(z, wa, ba, g, be, rm, rv, wb, bb):
    z = _dot(z, wa[...]) + ba[...]
    scale = g[...] * lax.rsqrt(rv[...] + 1e-5)
    z = (z - rm[...]) * scale + be[...]
    z = jnp.maximum(z, 0.0)
    z = _dot(z, wb[...]) + bb[...]
    return jnp.maximum(z, 0.0)


def _tc_layer1(h, p, Wa, ba, g, be, rm, rv, Wb, bb, blk=2000):
    N, D = h.shape
    grid = N // blk

    def body(h_ref, p_ref, wa, ba_r, g_r, be_r, rm_r, rv_r, wb, bb_r,
             o_ref):
        z = h_ref[...] + p_ref[0] + p_ref[1]
        o_ref[...] = _gin_mlp(z, wa, ba_r, g_r, be_r, rm_r, rv_r, wb, bb_r)

    return pl.pallas_call(
        body,
        grid=(grid,),
        in_specs=[pl.BlockSpec((blk, D), lambda i: (i, 0)),
                  pl.BlockSpec((2, blk, D), lambda i: (0, i, 0))]
        + [_mat_spec(D)] + [_row_spec(D)] * 5 + [_mat_spec(D), _row_spec(D)],
        out_specs=pl.BlockSpec((blk, D), lambda i: (i, 0)),
        out_shape=jax.ShapeDtypeStruct((N, D), jnp.float32),
    )(h, p, Wa, ba.reshape(1, D), g.reshape(1, D), be.reshape(1, D),
      rm.reshape(1, D), rv.reshape(1, D), Wb, bb.reshape(1, D))


def _tc_layer2_head(h, p, Wa, ba, g, be, rm, rv, Wb, bb,
                    W5, b5, W6, b6, blk=2000):
    N, D = h.shape
    OUT = W6.shape[1]
    grid = N // blk

    def body(h_ref, p_ref, wa, ba_r, g_r, be_r, rm_r, rv_r, wb, bb_r,
             w5, b5_r, w6, b6_r, o_ref, acc):
        i = pl.program_id(0)
        z = h_ref[...] + p_ref[0] + p_ref[1]
        z = _gin_mlp(z, wa, ba_r, g_r, be_r, rm_r, rv_r, wb, bb_r)
        psum = jnp.sum(z, axis=0, keepdims=True)

        @pl.when(i == 0)
        def _():
            acc[...] = psum

        @pl.when(i > 0)
        def _():
            acc[...] = acc[...] + psum

        @pl.when(i == grid - 1)
        def _():
            y = jnp.maximum(_dot(acc[...], w5[...]) + b5_r[...], 0.0)
            o_ref[...] = _dot(y, w6[...]) + b6_r[...]

    return pl.pallas_call(
        body,
        grid=(grid,),
        in_specs=[pl.BlockSpec((blk, D), lambda i: (i, 0)),
                  pl.BlockSpec((2, blk, D), lambda i: (0, i, 0))]
        + [_mat_spec(D)] + [_row_spec(D)] * 5 + [_mat_spec(D), _row_spec(D)]
        + [pl.BlockSpec((D, OUT), lambda i: (0, 0)),
           pl.BlockSpec((1, OUT), lambda i: (0, 0)),
           pl.BlockSpec((D, OUT), lambda i: (0, 0)),
           pl.BlockSpec((1, OUT), lambda i: (0, 0))],
        out_specs=pl.BlockSpec((1, OUT), lambda i: (0, 0)),
        out_shape=jax.ShapeDtypeStruct((1, OUT), jnp.float32),
        scratch_shapes=[pltpu.VMEM((1, D), jnp.float32)],
    )(h, p, Wa, ba.reshape(1, D), g.reshape(1, D), be.reshape(1, D),
      rm.reshape(1, D), rv.reshape(1, D), Wb, bb.reshape(1, D),
      W5, b5.reshape(1, OUT), W6, b6.reshape(1, OUT))


def kernel(x, edge_index, W1, b1, g1, be1, W2, b2, W3, b3, g2, be2,
           W4, b4, W5, b5, W6, b6, rm1, rv1, rm2, rv2):
    N, D = x.shape
    E = edge_index.shape[1]
    NW = _NC * _NS
    C = E // (NW * _K)
    src3 = edge_index[0].reshape(NW, C, _K)
    dst3 = edge_index[1].reshape(NW, C, _K)
    NP = (-(-N // (_NS * 8)) * 8) * _NS
    zeros_np = jnp.zeros((NP, D), jnp.float32)

    p1 = _sc_aggregate(x, src3, dst3, zeros_np)
    h1 = _tc_layer1(x, p1, W1, b1, g1, be1, rm1, rv1, W2, b2)
    p2 = _sc_aggregate(h1, src3, dst3, zeros_np)
    return _tc_layer2_head(h1, p2, W3, b3, g2, be2, rm2, rv2,
                           W4, b4, W5, b5, W6, b6)


# speedup vs baseline: 8.3472x; 1.2483x over previous
import functools

import jax
import jax.numpy as jnp
from jax import lax
from jax.experimental import pallas as pl
from jax.experimental.pallas import tpu as pltpu
from jax.experimental.pallas import tpu_sc as plsc

_NC = 2
_NS = 16
_K = 128
_G = 8


def _sc_aggregate(h, src3, dst3, zeros_np):
    N, D = h.shape
    NW, C, K = src3.shape
    ROWS = -(-N // (_NS * 8)) * 8
    NP = ROWS * _NS
    G = _G
    NGRP = C // G

    mesh = plsc.VectorSubcoreMesh(core_axis_name="c", subcore_axis_name="s")

    @functools.partial(
        pl.kernel,
        out_type=jax.ShapeDtypeStruct((_NC, NP, D), jnp.float32),
        mesh=mesh,
        scratch_types=[
            pltpu.VMEM_SHARED((NP, D), jnp.float32),
            pltpu.VMEM((G, K), jnp.int32),
            pltpu.VMEM((G, K), jnp.int32),
            pltpu.VMEM((K, D), jnp.float32),
            pltpu.VMEM((K, D), jnp.float32),
            pltpu.SemaphoreType.DMA,
            pltpu.SemaphoreType.DMA,
        ],
    )
    def agg_kernel(h_hbm, src_hbm, dst_hbm, zeros_hbm, out_hbm,
                   agg_sh, src_v, dst_v, rows0_v, rows1_v, sem0, sem1):
        cid = lax.axis_index("c")
        sid = lax.axis_index("s")
        wid = cid * _NS + sid

        base = sid * ROWS
        pltpu.sync_copy(zeros_hbm.at[pl.ds(base, ROWS)],
                        agg_sh.at[pl.ds(base, ROWS)])
        plsc.subcore_barrier()

        def gbody(g, carry):
            pltpu.sync_copy(src_hbm.at[wid, pl.ds(g * G, G)], src_v)
            pltpu.sync_copy(dst_hbm.at[wid, pl.ds(g * G, G)], dst_v)

            def ebody(i, carry2):
                j = i * 2
                d0 = pltpu.async_copy(h_hbm.at[src_v.at[j]], rows0_v, sem0)
                d1 = pltpu.async_copy(h_hbm.at[src_v.at[j + 1]], rows1_v,
                                      sem1)
                d0.wait()
                pltpu.sync_copy(rows0_v, agg_sh.at[dst_v.at[j]], add=True)
                d1.wait()
                pltpu.sync_copy(rows1_v, agg_sh.at[dst_v.at[j + 1]], add=True)
                return carry2

            lax.fori_loop(0, G // 2, ebody, 0)
            return carry

        lax.fori_loop(0, NGRP, gbody, 0)
        plsc.subcore_barrier()

        pltpu.sync_copy(agg_sh.at[pl.ds(base, ROWS)],
                        out_hbm.at[cid, pl.ds(base, ROWS)])

    return agg_kernel(h, src3, dst3, zeros_np)


def _row_spec(D):
    return pl.BlockSpec((1, D), lambda i: (0, 0))


def _mat_spec(D):
    return pl.BlockSpec((D, D), lambda i: (0, 0))


def _dot(a, b):
    return jnp.dot(a, b, preferred_element_type=jnp.float32)


def _gin_mlp(z, wa, ba, g, be, rm, rv, wb, bb):
    z = _dot(z, wa[...]) + ba[...]
    scale = g[...] * lax.rsqrt(rv[...] + 1e-5)
    z = (z - rm[...]) * scale + be[...]
    z = jnp.maximum(z, 0.0)
    z = _dot(z, wb[...]) + bb[...]
    return jnp.maximum(z, 0.0)


def _tc_layer1(h, p, Wa, ba, g, be, rm, rv, Wb, bb, blk=2000):
    N, D = h.shape
    grid = N // blk

    def body(h_ref, p_ref, wa, ba_r, g_r, be_r, rm_r, rv_r, wb, bb_r,
             o_ref):
        z = h_ref[...] + p_ref[0] + p_ref[1]
        o_ref[...] = _gin_mlp(z, wa, ba_r, g_r, be_r, rm_r, rv_r, wb, bb_r)

    return pl.pallas_call(
        body,
        grid=(grid,),
        in_specs=[pl.BlockSpec((blk, D), lambda i: (i, 0)),
                  pl.BlockSpec((2, blk, D), lambda i: (0, i, 0))]
        + [_mat_spec(D)] + [_row_spec(D)] * 5 + [_mat_spec(D), _row_spec(D)],
        out_specs=pl.BlockSpec((blk, D), lambda i: (i, 0)),
        out_shape=jax.ShapeDtypeStruct((N, D), jnp.float32),
    )(h, p, Wa, ba.reshape(1, D), g.reshape(1, D), be.reshape(1, D),
      rm.reshape(1, D), rv.reshape(1, D), Wb, bb.reshape(1, D))


def _tc_layer2_head(h, p, Wa, ba, g, be, rm, rv, Wb, bb,
                    W5, b5, W6, b6, blk=2000):
    N, D = h.shape
    OUT = W6.shape[1]
    grid = N // blk

    def body(h_ref, p_ref, wa, ba_r, g_r, be_r, rm_r, rv_r, wb, bb_r,
             w5, b5_r, w6, b6_r, o_ref, acc):
        i = pl.program_id(0)
        z = h_ref[...] + p_ref[0] + p_ref[1]
        z = _gin_mlp(z, wa, ba_r, g_r, be_r, rm_r, rv_r, wb, bb_r)
        psum = jnp.sum(z, axis=0, keepdims=True)

        @pl.when(i == 0)
        def _():
            acc[...] = psum

        @pl.when(i > 0)
        def _():
            acc[...] = acc[...] + psum

        @pl.when(i == grid - 1)
        def _():
            y = jnp.maximum(_dot(acc[...], w5[...]) + b5_r[...], 0.0)
            o_ref[...] = _dot(y, w6[...]) + b6_r[...]

    return pl.pallas_call(
        body,
        grid=(grid,),
        in_specs=[pl.BlockSpec((blk, D), lambda i: (i, 0)),
                  pl.BlockSpec((2, blk, D), lambda i: (0, i, 0))]
        + [_mat_spec(D)] + [_row_spec(D)] * 5 + [_mat_spec(D), _row_spec(D)]
        + [pl.BlockSpec((D, OUT), lambda i: (0, 0)),
           pl.BlockSpec((1, OUT), lambda i: (0, 0)),
           pl.BlockSpec((D, OUT), lambda i: (0, 0)),
           pl.BlockSpec((1, OUT), lambda i: (0, 0))],
        out_specs=pl.BlockSpec((1, OUT), lambda i: (0, 0)),
        out_shape=jax.ShapeDtypeStruct((1, OUT), jnp.float32),
        scratch_shapes=[pltpu.VMEM((1, D), jnp.float32)],
    )(h, p, Wa, ba.reshape(1, D), g.reshape(1, D), be.reshape(1, D),
      rm.reshape(1, D), rv.reshape(1, D), Wb, bb.reshape(1, D),
      W5, b5.reshape(1, OUT), W6, b6.reshape(1, OUT))


def kernel(x, edge_index, W1, b1, g1, be1, W2, b2, W3, b3, g2, be2,
           W4, b4, W5, b5, W6, b6, rm1, rv1, rm2, rv2):
    N, D = x.shape
    E = edge_index.shape[1]
    NW = _NC * _NS
    NP = (-(-N // (_NS * 8)) * 8) * _NS
    zeros_np = jnp.zeros((NP, D), jnp.float32)

    quant = NW * _K * _G
    EP = -(-E // quant) * quant
    pad = EP - E
    src = edge_index[0]
    dst = edge_index[1]
    if pad:
        ar = jnp.arange(pad, dtype=jnp.int32)
        src = jnp.concatenate([src, ar % N])
        dst = jnp.concatenate([dst, N + ar % (NP - N)])
    C = EP // (NW * _K)
    src3 = src.reshape(NW, C, _K)
    dst3 = dst.reshape(NW, C, _K)

    p1 = _sc_aggregate(x, src3, dst3, zeros_np)
    h1 = _tc_layer1(x, p1, W1, b1, g1, be1, rm1, rv1, W2, b2)
    p2 = _sc_aggregate(h1, src3, dst3, zeros_np)
    return _tc_layer2_head(h1, p2, W3, b3, g2, be2, rm2, rv2,
                           W4, b4, W5, b5, W6, b6)


# speedup vs baseline: 11.4822x; 1.3756x over previous
import functools

import jax
import jax.numpy as jnp
from jax import lax
from jax.experimental import pallas as pl
from jax.experimental.pallas import tpu as pltpu
from jax.experimental.pallas import tpu_sc as plsc

_NC = 2
_NS = 16
_K = 128
_G = 8


def _sc_aggregate(h, src3, dst3, zeros_np):
    N, D = h.shape
    NW, C, K = src3.shape
    ROWS = -(-N // (_NS * 8)) * 8
    NP = ROWS * _NS
    G = _G
    NGRP = C // G

    mesh = plsc.VectorSubcoreMesh(core_axis_name="c", subcore_axis_name="s")

    @functools.partial(
        pl.kernel,
        out_type=jax.ShapeDtypeStruct((_NC, NP, D), jnp.float32),
        mesh=mesh,
        scratch_types=[
            pltpu.VMEM_SHARED((NP, D), jnp.float32),
            pltpu.VMEM((G, K), jnp.int32),
            pltpu.VMEM((G, K), jnp.int32),
            pltpu.VMEM((G, K), jnp.int32),
            pltpu.VMEM((G, K), jnp.int32),
            pltpu.VMEM((K, D), jnp.float32),
            pltpu.VMEM((K, D), jnp.float32),
            pltpu.SemaphoreType.DMA,
            pltpu.SemaphoreType.DMA,
            pltpu.SemaphoreType.DMA,
            pltpu.SemaphoreType.DMA,
        ],
    )
    def agg_kernel(h_hbm, src_hbm, dst_hbm, zeros_hbm, out_hbm,
                   agg_sh, srcA, dstA, srcB, dstB, rows0_v, rows1_v,
                   sem0, sem1, semiA, semiB):
        cid = lax.axis_index("c")
        sid = lax.axis_index("s")
        wid = cid * _NS + sid

        base = sid * ROWS
        pltpu.sync_copy(zeros_hbm.at[pl.ds(base, ROWS)],
                        agg_sh.at[pl.ds(base, ROWS)])
        plsc.subcore_barrier()

        def wait_rows(sem):
            pltpu.make_async_copy(h_hbm.at[pl.ds(0, K)], rows0_v, sem).wait()

        def wait_idx(sem):
            pltpu.make_async_copy(src_hbm.at[wid, pl.ds(0, G)], srcA,
                                  sem).wait()

        def steady_pair(sv, dv, j):
            wait_rows(sem0)
            pltpu.sync_copy(rows0_v, agg_sh.at[dv.at[j]], add=True)
            pltpu.async_copy(h_hbm.at[sv.at[j + 2]], rows0_v, sem0)
            wait_rows(sem1)
            pltpu.sync_copy(rows1_v, agg_sh.at[dv.at[j + 1]], add=True)
            pltpu.async_copy(h_hbm.at[sv.at[j + 3]], rows1_v, sem1)

        def group_body(sv, dv, sv_next, dv_next, semi_next, gnext):
            pltpu.async_copy(src_hbm.at[wid, pl.ds(gnext * G, G)], sv_next,
                             semi_next)
            pltpu.async_copy(dst_hbm.at[wid, pl.ds(gnext * G, G)], dv_next,
                             semi_next)

            def ib(i, c):
                steady_pair(sv, dv, i * 2)
                return c

            lax.fori_loop(0, (G - 2) // 2, ib, 0)
            wait_idx(semi_next)
            wait_idx(semi_next)
            wait_rows(sem0)
            pltpu.sync_copy(rows0_v, agg_sh.at[dv.at[G - 2]], add=True)
            pltpu.async_copy(h_hbm.at[sv_next.at[0]], rows0_v, sem0)
            wait_rows(sem1)
            pltpu.sync_copy(rows1_v, agg_sh.at[dv.at[G - 1]], add=True)
            pltpu.async_copy(h_hbm.at[sv_next.at[1]], rows1_v, sem1)

        pltpu.sync_copy(src_hbm.at[wid, pl.ds(0, G)], srcA)
        pltpu.sync_copy(dst_hbm.at[wid, pl.ds(0, G)], dstA)
        pltpu.async_copy(h_hbm.at[srcA.at[0]], rows0_v, sem0)
        pltpu.async_copy(h_hbm.at[srcA.at[1]], rows1_v, sem1)

        def outer(m, carry):
            g = m * 2
            group_body(srcA, dstA, srcB, dstB, semiB, g + 1)
            group_body(srcB, dstB, srcA, dstA, semiA, lax.rem(g + 2, NGRP))
            return carry

        lax.fori_loop(0, NGRP // 2, outer, 0)
        wait_rows(sem0)
        wait_rows(sem1)
        plsc.subcore_barrier()

        pltpu.sync_copy(agg_sh.at[pl.ds(base, ROWS)],
                        out_hbm.at[cid, pl.ds(base, ROWS)])

    return agg_kernel(h, src3, dst3, zeros_np)


def _row_spec(D):
    return pl.BlockSpec((1, D), lambda i: (0, 0))


def _mat_spec(D):
    return pl.BlockSpec((D, D), lambda i: (0, 0))


def _dot(a, b):
    return jnp.dot(a, b, preferred_element_type=jnp.float32)


def _gin_mlp(z, wa, ba, g, be, rm, rv, wb, bb):
    z = _dot(z, wa[...]) + ba[...]
    scale = g[...] * lax.rsqrt(rv[...] + 1e-5)
    z = (z - rm[...]) * scale + be[...]
    z = jnp.maximum(z, 0.0)
    z = _dot(z, wb[...]) + bb[...]
    return jnp.maximum(z, 0.0)


def _tc_layer1(h, p, Wa, ba, g, be, rm, rv, Wb, bb, blk=2000):
    N, D = h.shape
    grid = N // blk

    def body(h_ref, p_ref, wa, ba_r, g_r, be_r, rm_r, rv_r, wb, bb_r,
             o_ref):
        z = h_ref[...] + p_ref[0] + p_ref[1]
        o_ref[...] = _gin_mlp(z, wa, ba_r, g_r, be_r, rm_r, rv_r, wb, bb_r)

    return pl.pallas_call(
        body,
        grid=(grid,),
        in_specs=[pl.BlockSpec((blk, D), lambda i: (i, 0)),
                  pl.BlockSpec((2, blk, D), lambda i: (0, i, 0))]
        + [_mat_spec(D)] + [_row_spec(D)] * 5 + [_mat_spec(D), _row_spec(D)],
        out_specs=pl.BlockSpec((blk, D), lambda i: (i, 0)),
        out_shape=jax.ShapeDtypeStruct((N, D), jnp.float32),
    )(h, p, Wa, ba.reshape(1, D), g.reshape(1, D), be.reshape(1, D),
      rm.reshape(1, D), rv.reshape(1, D), Wb, bb.reshape(1, D))


def _tc_layer2_head(h, p, Wa, ba, g, be, rm, rv, Wb, bb,
                    W5, b5, W6, b6, blk=2000):
    N, D = h.shape
    OUT = W6.shape[1]
    grid = N // blk

    def body(h_ref, p_ref, wa, ba_r, g_r, be_r, rm_r, rv_r, wb, bb_r,
             w5, b5_r, w6, b6_r, o_ref, acc):
        i = pl.program_id(0)
        z = h_ref[...] + p_ref[0] + p_ref[1]
        z = _gin_mlp(z, wa, ba_r, g_r, be_r, rm_r, rv_r, wb, bb_r)
        psum = jnp.sum(z, axis=0, keepdims=True)

        @pl.when(i == 0)
        def _():
            acc[...] = psum

        @pl.when(i > 0)
        def _():
            acc[...] = acc[...] + psum

        @pl.when(i == grid - 1)
        def _():
            y = jnp.maximum(_dot(acc[...], w5[...]) + b5_r[...], 0.0)
            o_ref[...] = _dot(y, w6[...]) + b6_r[...]

    return pl.pallas_call(
        body,
        grid=(grid,),
        in_specs=[pl.BlockSpec((blk, D), lambda i: (i, 0)),
                  pl.BlockSpec((2, blk, D), lambda i: (0, i, 0))]
        + [_mat_spec(D)] + [_row_spec(D)] * 5 + [_mat_spec(D), _row_spec(D)]
        + [pl.BlockSpec((D, OUT), lambda i: (0, 0)),
           pl.BlockSpec((1, OUT), lambda i: (0, 0)),
           pl.BlockSpec((D, OUT), lambda i: (0, 0)),
           pl.BlockSpec((1, OUT), lambda i: (0, 0))],
        out_specs=pl.BlockSpec((1, OUT), lambda i: (0, 0)),
        out_shape=jax.ShapeDtypeStruct((1, OUT), jnp.float32),
        scratch_shapes=[pltpu.VMEM((1, D), jnp.float32)],
    )(h, p, Wa, ba.reshape(1, D), g.reshape(1, D), be.reshape(1, D),
      rm.reshape(1, D), rv.reshape(1, D), Wb, bb.reshape(1, D),
      W5, b5.reshape(1, OUT), W6, b6.reshape(1, OUT))


def kernel(x, edge_index, W1, b1, g1, be1, W2, b2, W3, b3, g2, be2,
           W4, b4, W5, b5, W6, b6, rm1, rv1, rm2, rv2):
    N, D = x.shape
    E = edge_index.shape[1]
    NW = _NC * _NS
    NP = (-(-N // (_NS * 8)) * 8) * _NS
    zeros_np = jnp.zeros((NP, D), jnp.float32)

    quant = NW * _K * _G * 2
    EP = -(-E // quant) * quant
    pad = EP - E
    src = edge_index[0]
    dst = edge_index[1]
    if pad:
        ar = jnp.arange(pad, dtype=jnp.int32)
        src = jnp.concatenate([src, ar % N])
        dst = jnp.concatenate([dst, N + ar % (NP - N)])
    C = EP // (NW * _K)
    src3 = src.reshape(NW, C, _K)
    dst3 = dst.reshape(NW, C, _K)

    p1 = _sc_aggregate(x, src3, dst3, zeros_np)
    h1 = _tc_layer1(x, p1, W1, b1, g1, be1, rm1, rv1, W2, b2)
    p2 = _sc_aggregate(h1, src3, dst3, zeros_np)
    return _tc_layer2_head(h1, p2, W3, b3, g2, be2, rm2, rv2,
                           W4, b4, W5, b5, W6, b6)


# speedup vs baseline: 11.7989x; 1.0276x over previous
import functools

import jax
import jax.numpy as jnp
from jax import lax
from jax.experimental import pallas as pl
from jax.experimental.pallas import tpu as pltpu
from jax.experimental.pallas import tpu_sc as plsc

_NC = 2
_NS = 16
_K = 128
_G = 8


def _sc_aggregate(h, src3, dst3, zeros_np):
    N, D = h.shape
    NW, C, K = src3.shape
    ROWS = -(-N // (_NS * 8)) * 8
    NP = ROWS * _NS
    G = _G
    NGRP = C // G

    mesh = plsc.VectorSubcoreMesh(core_axis_name="c", subcore_axis_name="s")

    @functools.partial(
        pl.kernel,
        out_type=jax.ShapeDtypeStruct((_NC, NP, D), jnp.float32),
        mesh=mesh,
        scratch_types=[
            pltpu.VMEM_SHARED((NP, D), jnp.float32),
            pltpu.VMEM((G, K), jnp.int32),
            pltpu.VMEM((G, K), jnp.int32),
            pltpu.VMEM((G, K), jnp.int32),
            pltpu.VMEM((G, K), jnp.int32),
            pltpu.VMEM((K, D), jnp.float32),
            pltpu.VMEM((K, D), jnp.float32),
            pltpu.SemaphoreType.DMA,
            pltpu.SemaphoreType.DMA,
            pltpu.SemaphoreType.DMA,
            pltpu.SemaphoreType.DMA,
            pltpu.SemaphoreType.DMA,
        ],
    )
    def agg_kernel(h_hbm, src_hbm, dst_hbm, zeros_hbm, out_hbm,
                   agg_sh, srcA, dstA, srcB, dstB, rows0_v, rows1_v,
                   sem0, sem1, semiA, semiB, semz):
        cid = lax.axis_index("c")
        sid = lax.axis_index("s")
        wid = cid * _NS + sid

        base = sid * ROWS
        zd = pltpu.async_copy(zeros_hbm.at[pl.ds(base, ROWS)],
                              agg_sh.at[pl.ds(base, ROWS)], semz)

        def wait_rows(sem):
            pltpu.make_async_copy(h_hbm.at[pl.ds(0, K)], rows0_v, sem).wait()

        def wait_idx(sem):
            pltpu.make_async_copy(src_hbm.at[wid, pl.ds(0, G)], srcA,
                                  sem).wait()

        def steady_pair(sv, dv, j):
            wait_rows(sem0)
            pltpu.sync_copy(rows0_v, agg_sh.at[dv.at[j]], add=True)
            pltpu.async_copy(h_hbm.at[sv.at[j + 2]], rows0_v, sem0)
            wait_rows(sem1)
            pltpu.sync_copy(rows1_v, agg_sh.at[dv.at[j + 1]], add=True)
            pltpu.async_copy(h_hbm.at[sv.at[j + 3]], rows1_v, sem1)

        def group_body(sv, dv, sv_next, dv_next, semi_next, gnext):
            pltpu.async_copy(src_hbm.at[wid, pl.ds(gnext * G, G)], sv_next,
                             semi_next)
            pltpu.async_copy(dst_hbm.at[wid, pl.ds(gnext * G, G)], dv_next,
                             semi_next)

            def ib(i, c):
                steady_pair(sv, dv, i * 2)
                return c

            lax.fori_loop(0, (G - 2) // 2, ib, 0)
            wait_idx(semi_next)
            wait_idx(semi_next)
            wait_rows(sem0)
            pltpu.sync_copy(rows0_v, agg_sh.at[dv.at[G - 2]], add=True)
            pltpu.async_copy(h_hbm.at[sv_next.at[0]], rows0_v, sem0)
            wait_rows(sem1)
            pltpu.sync_copy(rows1_v, agg_sh.at[dv.at[G - 1]], add=True)
            pltpu.async_copy(h_hbm.at[sv_next.at[1]], rows1_v, sem1)

        pltpu.sync_copy(src_hbm.at[wid, pl.ds(0, G)], srcA)
        pltpu.sync_copy(dst_hbm.at[wid, pl.ds(0, G)], dstA)
        pltpu.async_copy(h_hbm.at[srcA.at[0]], rows0_v, sem0)
        pltpu.async_copy(h_hbm.at[srcA.at[1]], rows1_v, sem1)
        zd.wait()
        plsc.subcore_barrier()

        def outer(m, carry):
            g = m * 2
            group_body(srcA, dstA, srcB, dstB, semiB, g + 1)
            group_body(srcB, dstB, srcA, dstA, semiA, lax.rem(g + 2, NGRP))
            return carry

        lax.fori_loop(0, NGRP // 2, outer, 0)
        wait_rows(sem0)
        wait_rows(sem1)
        plsc.subcore_barrier()

        pltpu.sync_copy(agg_sh.at[pl.ds(base, ROWS)],
                        out_hbm.at[cid, pl.ds(base, ROWS)])

    return agg_kernel(h, src3, dst3, zeros_np)


def _row_spec(D):
    return pl.BlockSpec((1, D), lambda i: (0, 0))


def _mat_spec(D):
    return pl.BlockSpec((D, D), lambda i: (0, 0))


def _dot(a, b):
    return jnp.dot(a, b, preferred_element_type=jnp.float32)


def _gin_mlp(z, wa, ba, g, be, rm, rv, wb, bb):
    z = _dot(z, wa[...]) + ba[...]
    scale = g[...] * lax.rsqrt(rv[...] + 1e-5)
    z = (z - rm[...]) * scale + be[...]
    z = jnp.maximum(z, 0.0)
    z = _dot(z, wb[...]) + bb[...]
    return jnp.maximum(z, 0.0)


def _tc_layer1(h, p, Wa, ba, g, be, rm, rv, Wb, bb, blk=2000):
    N, D = h.shape
    grid = N // blk

    def body(h_ref, p_ref, wa, ba_r, g_r, be_r, rm_r, rv_r, wb, bb_r,
             o_ref):
        z = h_ref[...] + p_ref[0] + p_ref[1]
        o_ref[...] = _gin_mlp(z, wa, ba_r, g_r, be_r, rm_r, rv_r, wb, bb_r)

    return pl.pallas_call(
        body,
        grid=(grid,),
        in_specs=[pl.BlockSpec((blk, D), lambda i: (i, 0)),
                  pl.BlockSpec((2, blk, D), lambda i: (0, i, 0))]
        + [_mat_spec(D)] + [_row_spec(D)] * 5 + [_mat_spec(D), _row_spec(D)],
        out_specs=pl.BlockSpec((blk, D), lambda i: (i, 0)),
        out_shape=jax.ShapeDtypeStruct((N, D), jnp.float32),
    )(h, p, Wa, ba.reshape(1, D), g.reshape(1, D), be.reshape(1, D),
      rm.reshape(1, D), rv.reshape(1, D), Wb, bb.reshape(1, D))


def _tc_layer2_head(h, p, Wa, ba, g, be, rm, rv, Wb, bb,
                    W5, b5, W6, b6, blk=2000):
    N, D = h.shape
    OUT = W6.shape[1]
    grid = N // blk

    def body(h_ref, p_ref, wa, ba_r, g_r, be_r, rm_r, rv_r, wb, bb_r,
             w5, b5_r, w6, b6_r, o_ref, acc):
        i = pl.program_id(0)
        z = h_ref[...] + p_ref[0] + p_ref[1]
        z = _gin_mlp(z, wa, ba_r, g_r, be_r, rm_r, rv_r, wb, bb_r)
        psum = jnp.sum(z, axis=0, keepdims=True)

        @pl.when(i == 0)
        def _():
            acc[...] = psum

        @pl.when(i > 0)
        def _():
            acc[...] = acc[...] + psum

        @pl.when(i == grid - 1)
        def _():
            y = jnp.maximum(_dot(acc[...], w5[...]) + b5_r[...], 0.0)
            o_ref[...] = _dot(y, w6[...]) + b6_r[...]

    return pl.pallas_call(
        body,
        grid=(grid,),
        in_specs=[pl.BlockSpec((blk, D), lambda i: (i, 0)),
                  pl.BlockSpec((2, blk, D), lambda i: (0, i, 0))]
        + [_mat_spec(D)] + [_row_spec(D)] * 5 + [_mat_spec(D), _row_spec(D)]
        + [pl.BlockSpec((D, OUT), lambda i: (0, 0)),
           pl.BlockSpec((1, OUT), lambda i: (0, 0)),
           pl.BlockSpec((D, OUT), lambda i: (0, 0)),
           pl.BlockSpec((1, OUT), lambda i: (0, 0))],
        out_specs=pl.BlockSpec((1, OUT), lambda i: (0, 0)),
        out_shape=jax.ShapeDtypeStruct((1, OUT), jnp.float32),
        scratch_shapes=[pltpu.VMEM((1, D), jnp.float32)],
    )(h, p, Wa, ba.reshape(1, D), g.reshape(1, D), be.reshape(1, D),
      rm.reshape(1, D), rv.reshape(1, D), Wb, bb.reshape(1, D),
      W5, b5.reshape(1, OUT), W6, b6.reshape(1, OUT))


def kernel(x, edge_index, W1, b1, g1, be1, W2, b2, W3, b3, g2, be2,
           W4, b4, W5, b5, W6, b6, rm1, rv1, rm2, rv2):
    N, D = x.shape
    E = edge_index.shape[1]
    NW = _NC * _NS
    NP = (-(-N // (_NS * 8)) * 8) * _NS
    zeros_np = jnp.zeros((NP, D), jnp.float32)

    quant = NW * _K * _G * 2
    EP = -(-E // quant) * quant
    pad = EP - E
    src = edge_index[0]
    dst = edge_index[1]
    if pad:
        ar = jnp.arange(pad, dtype=jnp.int32)
        src = jnp.concatenate([src, ar % N])
        dst = jnp.concatenate([dst, N + ar % (NP - N)])
    C = EP // (NW * _K)
    src3 = src.reshape(NW, C, _K)
    dst3 = dst.reshape(NW, C, _K)

    p1 = _sc_aggregate(x, src3, dst3, zeros_np)
    h1 = _tc_layer1(x, p1, W1, b1, g1, be1, rm1, rv1, W2, b2)
    p2 = _sc_aggregate(h1, src3, dst3, zeros_np)
    return _tc_layer2_head(h1, p2, W3, b3, g2, be2, rm2, rv2,
                           W4, b4, W5, b5, W6, b6)
